# Initial kernel scaffold; baseline (speedup 1.0000x reference)
import jax
import jax.numpy as jnp
from jax.experimental import pallas as pl


def kernel(input, adj_indices, adj_values, weight, bias):
    raise NotImplementedError("write your pallas kernel here")


# speedup vs baseline: 4.5519x; 4.5519x over previous
import functools

import jax
import jax.numpy as jnp
from jax import lax
from jax.experimental import pallas as pl
from jax.experimental.pallas import tpu as pltpu
from jax.experimental.pallas import tpu_sc as plsc

N_NODES = 10000
F = 128
CH = 128
NC = 2
NS = 16
NW = NC * NS
STRIPE = 624
TAIL = N_NODES - NS * STRIPE


def _mm_body(x_ref, w_ref, o_ref):
    o_ref[...] = jnp.dot(x_ref[...], w_ref[...],
                         preferred_element_type=jnp.float32)


def _matmul(x, w):
    m = x.shape[0]
    bm = 1000
    grid = (m // bm,)
    return pl.pallas_call(
        _mm_body,
        grid=grid,
        in_specs=[
            pl.BlockSpec((bm, F), lambda i: (i, 0)),
            pl.BlockSpec((F, F), lambda i: (0, 0)),
        ],
        out_specs=pl.BlockSpec((bm, F), lambda i: (i, 0)),
        out_shape=jax.ShapeDtypeStruct((m, F), jnp.float32),
    )(x, w)


def _combine_body(p_ref, b_ref, o_ref):
    o_ref[...] = p_ref[0] + p_ref[1] + b_ref[...]


def _combine(parts, bias2d):
    m = parts.shape[1]
    bm = 1000
    grid = (m // bm,)
    return pl.pallas_call(
        _combine_body,
        grid=grid,
        in_specs=[
            pl.BlockSpec((NC, bm, F), lambda i: (0, i, 0)),
            pl.BlockSpec((1, F), lambda i: (0, 0)),
        ],
        out_specs=pl.BlockSpec((bm, F), lambda i: (i, 0)),
        out_shape=jax.ShapeDtypeStruct((m, F), jnp.float32),
    )(parts, bias2d)


def _sc_spmm(support, rows3, cols3, vals3, n_chunks):
    mesh = plsc.VectorSubcoreMesh(core_axis_name="c", subcore_axis_name="s")

    @functools.partial(
        pl.kernel,
        mesh=mesh,
        out_type=jax.ShapeDtypeStruct((NC, N_NODES, F), jnp.float32),
        scratch_types=[
            pltpu.VMEM((n_chunks, CH), jnp.int32),
            pltpu.VMEM((n_chunks, CH), jnp.int32),
            pltpu.VMEM((n_chunks, CH), jnp.float32),
            pltpu.VMEM((CH, F), jnp.float32),
            pltpu.VMEM_SHARED((N_NODES, F), jnp.float32),
            pltpu.SemaphoreType.DMA,
        ],
    )
    def k(support_hbm, rows_hbm, cols_hbm, vals_hbm, out_hbm,
          cols_v, rows_v, vals_v, gbuf, acc, sem):
        c = lax.axis_index("c")
        s = lax.axis_index("s")
        wid = c * NS + s

        pltpu.sync_copy(rows_hbm.at[wid], rows_v)
        pltpu.sync_copy(cols_hbm.at[wid], cols_v)
        pltpu.sync_copy(vals_hbm.at[wid], vals_v)

        zeros16 = jnp.zeros((16,), jnp.float32)

        def zrow(e, carry):
            for j in range(F // 16):
                gbuf[e, pl.ds(j * 16, 16)] = zeros16
            return carry

        lax.fori_loop(0, CH, zrow, 0)
        base = s * STRIPE
        for t in range(STRIPE // CH):
            pltpu.sync_copy(gbuf, acc.at[pl.ds(base + t * CH, CH)])
        rem = STRIPE % CH
        if rem:
            pltpu.sync_copy(gbuf.at[pl.ds(0, rem)],
                            acc.at[pl.ds(base + STRIPE - rem, rem)])

        @pl.when(s == 0)
        def _():
            pltpu.sync_copy(gbuf.at[pl.ds(0, TAIL)],
                            acc.at[pl.ds(NS * STRIPE, TAIL)])

        plsc.subcore_barrier()

        def chunk(kk, carry):
            pltpu.async_copy(support_hbm.at[cols_v.at[kk]], gbuf, sem).wait()

            def scale(g, cc):
                vv = vals_v[kk, pl.ds(g * 16, 16)]
                for lane in range(16):
                    sv = jnp.full((16,), vv[lane], jnp.float32)
                    e = g * 16 + lane
                    for j in range(F // 16):
                        sl = pl.ds(j * 16, 16)
                        gbuf[e, sl] = gbuf[e, sl] * sv
                return cc

            lax.fori_loop(0, CH // 16, scale, 0)
            pltpu.sync_copy(gbuf, acc.at[rows_v.at[kk]], add=True)
            return carry

        lax.fori_loop(0, n_chunks, chunk, 0)
        plsc.subcore_barrier()

        pltpu.sync_copy(acc.at[pl.ds(base, STRIPE)],
                        out_hbm.at[c, pl.ds(base, STRIPE)])

        @pl.when(s == 0)
        def _():
            pltpu.sync_copy(acc.at[pl.ds(NS * STRIPE, TAIL)],
                            out_hbm.at[c, pl.ds(NS * STRIPE, TAIL)])

    return k(support, rows3, cols3, vals3)


def kernel(input, adj_indices, adj_values, weight, bias):
    support = _matmul(input, weight)

    rows = adj_indices[0].astype(jnp.int32)
    cols = adj_indices[1].astype(jnp.int32)
    vals = adj_values.astype(jnp.float32)

    n_edges = vals.shape[0]
    per = -(-n_edges // (NW * CH)) * CH
    n_chunks = per // CH
    e_pad = per * NW
    pad = e_pad - n_edges
    rows = jnp.pad(rows, (0, pad))
    cols = jnp.pad(cols, (0, pad))
    vals = jnp.pad(vals, (0, pad))
    rows3 = rows.reshape(NW, n_chunks, CH)
    cols3 = cols.reshape(NW, n_chunks, CH)
    vals3 = vals.reshape(NW, n_chunks, CH)

    parts = _sc_spmm(support, rows3, cols3, vals3, n_chunks)
    return _combine(parts, bias.reshape(1, F))
